# baseline (speedup 1.0000x reference)
import functools

import jax
import jax.numpy as jnp
from jax import lax
from jax.experimental import pallas as pl
from jax.experimental.pallas import tpu as pltpu
from jax.experimental.pallas import tpu_sc as plsc

NUM_CORES = 2
NUM_SUBCORES = 16
NW = NUM_CORES * NUM_SUBCORES
LANES = 16


@functools.lru_cache(maxsize=None)
def _make_embed(n_rows, vocab, maxlen, embed, chunk_rows, interpret=False):
    assert n_rows % (NW * chunk_rows) == 0
    assert chunk_rows % maxlen == 0
    assert embed % LANES == 0
    rows_per_w = n_rows // NW
    n_chunks = rows_per_w // chunk_rows
    mesh = plsc.VectorSubcoreMesh(
        core_axis_name="c", subcore_axis_name="s",
        num_cores=NUM_CORES, num_subcores=NUM_SUBCORES)

    @functools.partial(
        pl.kernel,
        out_type=jax.ShapeDtypeStruct((n_rows, embed), jnp.float32),
        mesh=mesh,
        scratch_types=[
            pltpu.VMEM((chunk_rows,), jnp.int32),
            pltpu.VMEM((chunk_rows, embed), jnp.float32),
            pltpu.VMEM((maxlen, embed), jnp.float32),
            pltpu.SemaphoreType.DMA,
        ],
        compiler_params=pltpu.CompilerParams(use_tc_tiling_on_sc=False),
        interpret=interpret,
    )
    def embed_kernel(x_hbm, tok_hbm, pos_hbm, out_hbm, idx_v, rows_v, pos_v,
                     sem):
        wid = lax.axis_index("s") * NUM_CORES + lax.axis_index("c")
        base = wid * rows_per_w
        pltpu.sync_copy(pos_hbm, pos_v)

        def chunk_body(g, carry):
            cb = base + g * chunk_rows
            pltpu.sync_copy(x_hbm.at[pl.ds(cb, chunk_rows)], idx_v)
            pltpu.async_copy(tok_hbm.at[idx_v], rows_v, sem).wait()

            def row_body(j, carry2):
                sj = lax.rem(j, maxlen)
                for k in range(embed // LANES):
                    sl = pl.ds(k * LANES, LANES)
                    rows_v[j, sl] = rows_v[j, sl] + pos_v[sj, sl]
                return carry2

            lax.fori_loop(0, chunk_rows, row_body, 0, unroll=2)
            pltpu.sync_copy(rows_v, out_hbm.at[pl.ds(cb, chunk_rows)])
            return carry

        lax.fori_loop(0, n_chunks, chunk_body, 0)

    return embed_kernel


def kernel(x, token_table, pos_table):
    batch, seq = x.shape
    vocab, embed = token_table.shape
    maxlen = pos_table.shape[0]
    n_rows = batch * seq
    chunk_rows = 2 * maxlen
    fn = _make_embed(n_rows, vocab, maxlen, embed, chunk_rows)
    xf = x.reshape(n_rows).astype(jnp.int32)
    out = fn(xf, token_table, pos_table)
    return out.reshape(batch, seq, embed)


# speedup vs baseline: 1.2555x; 1.2555x over previous
import functools

import jax
import jax.numpy as jnp
from jax import lax
from jax.experimental import pallas as pl
from jax.experimental.pallas import tpu as pltpu
from jax.experimental.pallas import tpu_sc as plsc

NUM_CORES = 2
NUM_SUBCORES = 16
NW = NUM_CORES * NUM_SUBCORES
LANES = 16


@functools.lru_cache(maxsize=None)
def _make_embed(n_rows, vocab, maxlen, embed, chunk_rows, interpret=False):
    assert n_rows % (NW * chunk_rows) == 0
    assert chunk_rows % maxlen == 0
    assert embed % LANES == 0
    rows_per_w = n_rows // NW
    n_chunks = rows_per_w // chunk_rows
    assert n_chunks % 2 == 0
    reps = chunk_rows // maxlen
    groups = embed // LANES
    mesh = plsc.VectorSubcoreMesh(
        core_axis_name="c", subcore_axis_name="s",
        num_cores=NUM_CORES, num_subcores=NUM_SUBCORES)

    @functools.partial(
        pl.kernel,
        out_type=jax.ShapeDtypeStruct((n_rows, embed), jnp.float32),
        mesh=mesh,
        scratch_types=[
            pltpu.VMEM((rows_per_w,), jnp.int32),
            pltpu.VMEM((chunk_rows, embed), jnp.float32),
            pltpu.VMEM((chunk_rows, embed), jnp.float32),
            pltpu.VMEM((maxlen, embed), jnp.float32),
            pltpu.SemaphoreType.DMA,
            pltpu.SemaphoreType.DMA,
            pltpu.SemaphoreType.DMA,
            pltpu.SemaphoreType.DMA,
        ],
        compiler_params=pltpu.CompilerParams(use_tc_tiling_on_sc=False),
        interpret=interpret,
    )
    def embed_kernel(x_hbm, tok_hbm, pos_hbm, out_hbm, idx_v, rows0, rows1,
                     pos_v, sg0, sg1, so0, so1):
        wid = lax.axis_index("s") * NUM_CORES + lax.axis_index("c")
        base = wid * rows_per_w
        rows = (rows0, rows1)
        sg = (sg0, sg1)
        so = (so0, so1)

        pltpu.sync_copy(pos_hbm, pos_v)
        pltpu.sync_copy(x_hbm.at[pl.ds(base, rows_per_w)], idx_v)

        def gather(g, b):
            pltpu.async_copy(
                tok_hbm.at[idx_v.at[pl.ds(g * chunk_rows, chunk_rows)]],
                rows[b], sg[b])

        def wait_gather(b):
            pltpu.make_async_copy(
                tok_hbm.at[idx_v.at[pl.ds(0, chunk_rows)]], rows[b],
                sg[b]).wait()

        def put_out(g, b):
            pltpu.async_copy(
                rows[b], out_hbm.at[pl.ds(base + g * chunk_rows, chunk_rows)],
                so[b])

        def wait_out(b):
            pltpu.make_async_copy(
                rows[b], out_hbm.at[pl.ds(0, chunk_rows)], so[b]).wait()

        gather(0, 0)

        def step(g, b):
            wait_gather(b)

            @pl.when(g + 1 < n_chunks)
            def _():
                @pl.when(g >= 1)
                def _():
                    wait_out(1 - b)

                gather(g + 1, 1 - b)

            def add_body(s, carry):
                for k in range(groups):
                    sl = pl.ds(k * LANES, LANES)
                    p = pos_v[s, sl]
                    for r in range(reps):
                        j = s + r * maxlen
                        rows[b][j, sl] = rows[b][j, sl] + p
                return carry

            lax.fori_loop(0, maxlen, add_body, 0, unroll=2)
            put_out(g, b)
            return b

        def pair(g0, carry):
            step(g0 * 2, 0)
            step(g0 * 2 + 1, 1)
            return carry

        lax.fori_loop(0, n_chunks // 2, pair, 0)
        wait_out(0)
        wait_out(1)

    return embed_kernel


def kernel(x, token_table, pos_table):
    batch, seq = x.shape
    vocab, embed = token_table.shape
    maxlen = pos_table.shape[0]
    n_rows = batch * seq
    chunk_rows = 2 * maxlen
    fn = _make_embed(n_rows, vocab, maxlen, embed, chunk_rows)
    xf = x.reshape(n_rows).astype(jnp.int32)
    out = fn(xf, token_table, pos_table)
    return out.reshape(batch, seq, embed)


# speedup vs baseline: 1.2561x; 1.0004x over previous
import functools

import jax
import jax.numpy as jnp
from jax import lax
from jax.experimental import pallas as pl
from jax.experimental.pallas import tpu as pltpu
from jax.experimental.pallas import tpu_sc as plsc

NUM_CORES = 2
NUM_SUBCORES = 16
NW = NUM_CORES * NUM_SUBCORES
LANES = 16


@functools.lru_cache(maxsize=None)
def _make_embed(n_rows, vocab, maxlen, embed, chunk_rows, interpret=False):
    assert n_rows % (NW * chunk_rows) == 0
    assert chunk_rows % maxlen == 0
    assert embed % LANES == 0
    rows_per_w = n_rows // NW
    n_chunks = rows_per_w // chunk_rows
    assert n_chunks % 2 == 0
    reps = chunk_rows // maxlen
    groups = embed // LANES
    n_sub = 10
    assert chunk_rows % n_sub == 0
    assert (chunk_rows // n_sub) % 8 == 0
    mesh = plsc.VectorSubcoreMesh(
        core_axis_name="c", subcore_axis_name="s",
        num_cores=NUM_CORES, num_subcores=NUM_SUBCORES)

    @functools.partial(
        pl.kernel,
        out_type=jax.ShapeDtypeStruct((n_rows, embed), jnp.float32),
        mesh=mesh,
        scratch_types=[
            pltpu.VMEM((rows_per_w,), jnp.int32),
            pltpu.VMEM((chunk_rows, embed), jnp.float32),
            pltpu.VMEM((chunk_rows, embed), jnp.float32),
            pltpu.VMEM((maxlen, embed), jnp.float32),
            pltpu.SemaphoreType.DMA,
            pltpu.SemaphoreType.DMA,
            pltpu.SemaphoreType.DMA,
            pltpu.SemaphoreType.DMA,
        ],
        compiler_params=pltpu.CompilerParams(use_tc_tiling_on_sc=False),
        interpret=interpret,
    )
    def embed_kernel(x_hbm, tok_hbm, pos_hbm, out_hbm, idx_v, rows0, rows1,
                     pos_v, sg0, sg1, so0, so1):
        wid = lax.axis_index("s") * NUM_CORES + lax.axis_index("c")
        base = wid * rows_per_w
        rows = (rows0, rows1)
        sg = (sg0, sg1)
        so = (so0, so1)

        pltpu.sync_copy(pos_hbm, pos_v)
        pltpu.sync_copy(x_hbm.at[pl.ds(base, rows_per_w)], idx_v)

        sub_rows = chunk_rows // n_sub

        def gather(g, b):
            for i in range(n_sub):
                pltpu.async_copy(
                    tok_hbm.at[idx_v.at[pl.ds(
                        g * chunk_rows + i * sub_rows, sub_rows)]],
                    rows[b].at[pl.ds(i * sub_rows, sub_rows)], sg[b])

        def wait_gather(b):
            for i in range(n_sub):
                pltpu.make_async_copy(
                    tok_hbm.at[idx_v.at[pl.ds(0, sub_rows)]],
                    rows[b].at[pl.ds(i * sub_rows, sub_rows)], sg[b]).wait()

        def put_out(g, b):
            pltpu.async_copy(
                rows[b], out_hbm.at[pl.ds(base + g * chunk_rows, chunk_rows)],
                so[b])

        def wait_out(b):
            pltpu.make_async_copy(
                rows[b], out_hbm.at[pl.ds(0, chunk_rows)], so[b]).wait()

        gather(0, 0)

        def step(g, b):
            wait_gather(b)

            @pl.when(g + 1 < n_chunks)
            def _():
                @pl.when(g >= 1)
                def _():
                    wait_out(1 - b)

                gather(g + 1, 1 - b)

            def add_body(s, carry):
                for k in range(groups):
                    sl = pl.ds(k * LANES, LANES)
                    p = pos_v[s, sl]
                    for r in range(reps):
                        j = s + r * maxlen
                        rows[b][j, sl] = rows[b][j, sl] + p
                return carry

            lax.fori_loop(0, maxlen, add_body, 0, unroll=2)
            put_out(g, b)
            return b

        def pair(g0, carry):
            step(g0 * 2, 0)
            step(g0 * 2 + 1, 1)
            return carry

        lax.fori_loop(0, n_chunks // 2, pair, 0)
        wait_out(0)
        wait_out(1)

    return embed_kernel


def kernel(x, token_table, pos_table):
    batch, seq = x.shape
    vocab, embed = token_table.shape
    maxlen = pos_table.shape[0]
    n_rows = batch * seq
    chunk_rows = 2 * maxlen
    fn = _make_embed(n_rows, vocab, maxlen, embed, chunk_rows)
    xf = x.reshape(n_rows).astype(jnp.int32)
    out = fn(xf, token_table, pos_table)
    return out.reshape(batch, seq, embed)
